# Initial kernel scaffold; baseline (speedup 1.0000x reference)
import jax
import jax.numpy as jnp
from jax.experimental import pallas as pl


def kernel(feat_s, feat_q, support_labels):
    raise NotImplementedError("write your pallas kernel here")


# speedup vs baseline: 1.6635x; 1.6635x over previous
import functools

import jax
import jax.numpy as jnp
from jax import lax
from jax.experimental import pallas as pl


NUM_CLASSES = 64


def _protos_kernel(s_ref, lab_ref, pn_ref, pb2_ref, psum_ref):
    labels = lab_ref[...]
    classes = lax.broadcasted_iota(jnp.int32, (NUM_CLASSES, labels.shape[1]), 0)
    onehot = (labels == classes).astype(jnp.float32)
    sums = jnp.dot(onehot, s_ref[...], preferred_element_type=jnp.float32)
    counts = jnp.sum(onehot, axis=1, keepdims=True)
    protos = sums / jnp.maximum(counts, 1.0)
    psum_ref[...] = jnp.sum(protos, axis=0, keepdims=True)
    norm = jnp.sqrt(jnp.sum(protos * protos, axis=1, keepdims=True))
    pn = protos / jnp.maximum(norm, 1e-12)
    pn_ref[...] = pn
    pb2_ref[...] = jnp.sum(pn * pn, axis=1)[None, :]


def _degrees_kernel(q_ref, pn_ref, pb2_ref, deg_ref):
    q = q_ref[...]
    q2 = jnp.sum(q * q, axis=1, keepdims=True)
    inv_norm = lax.rsqrt(jnp.maximum(q2, 1e-24))
    cos = lax.dot_general(q, pn_ref[...], (((1,), (1,)), ((), ()))) * inv_norm
    d2 = 1.0 + pb2_ref[...] - 2.0 * cos
    deg = jnp.sum(jnp.sqrt(jnp.maximum(d2, 1e-12)), axis=1)
    deg_ref[...] = deg[None, None, :]


def _select_kernel(deg_ref, mask_ref, *, k):
    bits = lax.bitcast_convert_type(deg_ref[...], jnp.int32)

    def body(_, carry):
        lo, hi = carry
        mid = lo + (hi - lo + 1) // 2
        cnt = jnp.sum((bits >= mid).astype(jnp.int32))
        ok = cnt >= k
        return jnp.where(ok, mid, lo), jnp.where(ok, hi, mid - 1)

    lo, _ = lax.fori_loop(0, 31, body, (jnp.int32(0), jnp.int32(0x7F800000)))
    gt = bits > lo
    eq = bits == lo
    m = k - jnp.sum(gt.astype(jnp.int32))

    n = bits.shape[0]
    eqf = eq.astype(jnp.float32)
    i_idx = lax.broadcasted_iota(jnp.int32, (n, n), 0)
    j_idx = lax.broadcasted_iota(jnp.int32, (n, n), 1)
    lower_strict = (j_idx < i_idx).astype(jnp.float32)
    upper_strict = (i_idx < j_idx).astype(jnp.float32)
    row_tot = jnp.sum(eqf, axis=1, keepdims=True)
    row_excl = jnp.dot(lower_strict, row_tot, preferred_element_type=jnp.float32)
    col_excl = jnp.dot(eqf, upper_strict, preferred_element_type=jnp.float32)
    prefix = (row_excl + col_excl).astype(jnp.int32)
    mask_ref[...] = jnp.where(gt | (eq & (prefix < m)), 1.0, 0.0)


def _masked_sum_kernel(mask_ref, q_ref, psum_ref, mean_ref, *, denom):
    i = pl.program_id(0)

    @pl.when(i == 0)
    def _():
        mean_ref[...] = jnp.zeros_like(mean_ref)

    mean_ref[...] += jnp.dot(mask_ref[0], q_ref[...],
                             preferred_element_type=jnp.float32)

    @pl.when(i == pl.num_programs(0) - 1)
    def _():
        mean_ref[...] = (mean_ref[...] + psum_ref[...]) * (1.0 / denom)


def _sub_kernel(x_ref, mean_ref, out_ref):
    out_ref[...] = x_ref[...] - mean_ref[...]


def kernel(feat_s, feat_q, support_labels):
    S, D = feat_s.shape
    Q = feat_q.shape[0]
    C = NUM_CLASSES
    k = min(Q, max(S, Q // 4))

    labels = support_labels.astype(jnp.int32).reshape(1, S)

    pn, pb2, psum = pl.pallas_call(
        _protos_kernel,
        out_shape=(
            jax.ShapeDtypeStruct((C, D), jnp.float32),
            jax.ShapeDtypeStruct((1, C), jnp.float32),
            jax.ShapeDtypeStruct((1, D), jnp.float32),
        ),
    )(feat_s, labels)

    QB = 2048
    nq = Q // QB
    deg = pl.pallas_call(
        _degrees_kernel,
        grid=(nq,),
        in_specs=[
            pl.BlockSpec((QB, D), lambda i: (i, 0)),
            pl.BlockSpec((C, D), lambda i: (0, 0)),
            pl.BlockSpec((1, C), lambda i: (0, 0)),
        ],
        out_specs=pl.BlockSpec((1, 1, QB), lambda i: (i, 0, 0)),
        out_shape=jax.ShapeDtypeStruct((nq, 1, QB), jnp.float32),
    )(feat_q, pn, pb2)

    R = 128
    deg_sq = deg.reshape(R, R)
    mask = pl.pallas_call(
        functools.partial(_select_kernel, k=k),
        out_shape=jax.ShapeDtypeStruct((R, R), jnp.float32),
    )(deg_sq)

    mask3 = mask.reshape(nq, 1, QB)
    mean = pl.pallas_call(
        functools.partial(_masked_sum_kernel, denom=float(C + k)),
        grid=(nq,),
        in_specs=[
            pl.BlockSpec((1, 1, QB), lambda i: (i, 0, 0)),
            pl.BlockSpec((QB, D), lambda i: (i, 0)),
            pl.BlockSpec((1, D), lambda i: (0, 0)),
        ],
        out_specs=pl.BlockSpec((1, D), lambda i: (0, 0)),
        out_shape=jax.ShapeDtypeStruct((1, D), jnp.float32),
    )(mask3, feat_q, psum)

    out_s = pl.pallas_call(
        _sub_kernel,
        out_shape=jax.ShapeDtypeStruct((S, D), jnp.float32),
    )(feat_s, mean)

    out_q = pl.pallas_call(
        _sub_kernel,
        grid=(nq,),
        in_specs=[
            pl.BlockSpec((QB, D), lambda i: (i, 0)),
            pl.BlockSpec((1, D), lambda i: (0, 0)),
        ],
        out_specs=pl.BlockSpec((QB, D), lambda i: (i, 0)),
        out_shape=jax.ShapeDtypeStruct((Q, D), jnp.float32),
    )(feat_q, mean)

    return out_s, out_q
